# baseline (speedup 1.0000x reference)
import functools

import jax
import jax.numpy as jnp
from jax import lax
from jax.experimental import pallas as pl
from jax.experimental.pallas import tpu as pltpu
from jax.experimental.pallas import tpu_sc as plsc

N = 10000
NPAD = 10240
E = 320000
HID = 64
EDIM = 16
EPS = 1e-5

NBLK = 128
EBLK = 512

NWORK = 32
CH = 128
CPW = 80
EPAD = NWORK * CPW * CH


def _sc_gather_add(Ad, As, dst2d, src2d):
    mesh = plsc.VectorSubcoreMesh(core_axis_name="c", subcore_axis_name="s")

    @functools.partial(
        pl.kernel,
        out_type=jax.ShapeDtypeStruct((EPAD, HID), jnp.float32),
        mesh=mesh,
        scratch_types=[
            pltpu.VMEM((CPW, CH), jnp.int32),
            pltpu.VMEM((CPW, CH), jnp.int32),
            pltpu.VMEM((CH, HID), jnp.float32),
            pltpu.VMEM((CH, HID), jnp.float32),
            pltpu.SemaphoreType.DMA,
            pltpu.SemaphoreType.DMA,
        ],
        compiler_params=pltpu.CompilerParams(use_tc_tiling_on_sc=False),
    )
    def k(ad_hbm, as_hbm, d_hbm, s_hbm, out_hbm, didx, sidx, bufa, bufb,
          sema, semb):
        wid = lax.axis_index("s") * 2 + lax.axis_index("c")
        row0 = wid * CPW
        pltpu.sync_copy(d_hbm.at[pl.ds(row0, CPW)], didx)
        pltpu.sync_copy(s_hbm.at[pl.ds(row0, CPW)], sidx)

        def chunk_body(j, carry):
            ca = pltpu.async_copy(ad_hbm.at[didx.at[j]], bufa, sema)
            cb = pltpu.async_copy(as_hbm.at[sidx.at[j]], bufb, semb)
            ca.wait()
            cb.wait()

            def add_body(i, c2):
                for c in range(HID // 16):
                    sl = pl.ds(c * 16, 16)
                    bufa[i, sl] = bufa[i, sl] + bufb[i, sl]
                return c2

            lax.fori_loop(0, CH, add_body, 0, unroll=2)
            pltpu.sync_copy(bufa, out_hbm.at[pl.ds((row0 + j) * CH, CH)])
            return carry

        lax.fori_loop(0, CPW, chunk_body, 0)

    return k(Ad, As, dst2d, src2d)


NPW = 320
EPW = EPAD // NWORK
RROW = EPW + 128
_MAGIC = 209716


def _sc_route(dst2d):
    mesh = plsc.VectorSubcoreMesh(core_axis_name="c", subcore_axis_name="s")

    @functools.partial(
        pl.kernel,
        out_type=[
            jax.ShapeDtypeStruct((NWORK * RROW,), jnp.int32),
            jax.ShapeDtypeStruct((NWORK * NWORK,), jnp.int32),
        ],
        mesh=mesh,
        scratch_types=[
            pltpu.VMEM((CPW, CH), jnp.int32),
            pltpu.VMEM((EPW,), jnp.int32),
            pltpu.VMEM((NWORK,), jnp.int32),
            pltpu.VMEM((NWORK,), jnp.int32),
            pltpu.VMEM((NWORK,), jnp.int32),
        ],
        compiler_params=pltpu.CompilerParams(needs_layout_passes=False),
    )
    def k(d_hbm, routed_hbm, ends_hbm, didx, pend, cnt, cur, endsv):
        wid = lax.axis_index("s") * 2 + lax.axis_index("c")
        pltpu.sync_copy(d_hbm.at[pl.ds(wid * CPW, CPW)], didx)
        beta, _ = plsc.scan_count(lax.iota(jnp.int32, 16))
        lanes = lax.iota(jnp.int32, 16)
        lane0m = lanes == 0
        zeros = jnp.zeros((16,), jnp.int32)
        cnt[pl.ds(0, 16)] = zeros
        cnt[pl.ds(16, 16)] = zeros

        def count_row(r, c):
            drow = didx.at[r]
            for gg in range(CH // 16):
                v = drow[pl.ds(gg * 16, 16)]
                b = (v * _MAGIC) >> 26
                rank, last = plsc.scan_count(b)
                base = plsc.load_gather(cnt, [b])
                plsc.store_scatter(cnt, [b], base + (rank - beta) + 1,
                                   mask=last)
            return c

        lax.fori_loop(0, CPW, count_row, 0)

        c0 = cnt[pl.ds(0, 16)]
        c1 = cnt[pl.ds(16, 16)]
        run = jnp.int32(0)
        for kk in range(NWORK):
            ck = c0[kk] if kk < 16 else c1[kk - 16]
            kv = jnp.full((16,), kk, jnp.int32)
            plsc.store_scatter(cur, [kv], jnp.full((16,), run, jnp.int32),
                               mask=lane0m)
            run = run + ck
            plsc.store_scatter(endsv, [kv], jnp.full((16,), run, jnp.int32),
                               mask=lane0m)
        pltpu.sync_copy(
            endsv, ends_hbm.at[pl.ds(pl.multiple_of(wid * NWORK, 8), NWORK)])

        def place_row(r, c):
            drow = didx.at[r]
            for gg in range(CH // 16):
                v = drow[pl.ds(gg * 16, 16)]
                b = (v * _MAGIC) >> 26
                off = v - b * NPW
                eid = (wid * CPW + r) * CH + gg * 16 + lanes
                pk = (off << 19) | eid
                rank, last = plsc.scan_count(b)
                base = plsc.load_gather(cur, [b])
                pos = base + (rank - beta)
                plsc.store_scatter(pend, [pos], pk)
                plsc.store_scatter(cur, [b], pos + 1, mask=last)
            return c

        lax.fori_loop(0, CPW, place_row, 0)
        pltpu.sync_copy(
            pend, routed_hbm.at[pl.ds(pl.multiple_of(wid * RROW, 8), EPW)])

    return k(dst2d)


def _sc_segmax(M, routed, ends):
    mesh = plsc.VectorSubcoreMesh(core_axis_name="c", subcore_axis_name="s")

    @functools.partial(
        pl.kernel,
        out_type=jax.ShapeDtypeStruct((NPAD, HID), jnp.float32),
        mesh=mesh,
        scratch_types=[
            pltpu.VMEM((NPW, HID), jnp.float32),
            pltpu.VMEM((128, HID), jnp.float32),
            pltpu.VMEM((128,), jnp.int32),
            pltpu.VMEM((128,), jnp.int32),
            pltpu.VMEM((NWORK * NWORK,), jnp.int32),
            pltpu.SemaphoreType.DMA,
        ],
        compiler_params=pltpu.CompilerParams(
            needs_layout_passes=False, use_tc_tiling_on_sc=False),
    )
    def k(m_hbm, routed_hbm, ends_hbm, out_hbm, accum, mbuf, pkbuf, eidbuf,
          endsv, sem):
        o = lax.axis_index("s") * 2 + lax.axis_index("c")
        pltpu.sync_copy(ends_hbm, endsv)
        ninf = jnp.full((16,), -jnp.inf, jnp.float32)

        def init(r, c):
            arow = accum.at[r]
            for cc in range(HID // 16):
                arow[pl.ds(cc * 16, 16)] = ninf
            return c

        lax.fori_loop(0, NPW, init, 0)
        ov = jnp.full((16,), o, jnp.int32)
        ovm1 = jnp.full((16,), jnp.maximum(o - 1, 0), jnp.int32)

        def per_w(w, c):
            wv = jnp.full((16,), w * NWORK, jnp.int32)
            e_s = plsc.load_gather(endsv, [wv + ov])[0]
            s_raw = plsc.load_gather(endsv, [wv + ovm1])[0]
            s_s = jnp.where(o > 0, s_raw, 0)
            c_s = e_s - s_s
            s8 = s_s & (-8)
            npc = (s_s - s8 + c_s + 127) // 128

            def piece(p, c2):
                basepos = s8 + p * 128
                pltpu.sync_copy(
                    routed_hbm.at[
                        pl.ds(pl.multiple_of(w * RROW + basepos, 8), 128)],
                    pkbuf)
                for gg in range(8):
                    pkv = pkbuf[pl.ds(gg * 16, 16)]
                    eidbuf[pl.ds(gg * 16, 16)] = jnp.minimum(
                        pkv & 0x7FFFF, EPAD - 1)
                pltpu.async_copy(m_hbm.at[eidbuf], mbuf, sem).wait()
                for gg in range(8):
                    pkv = pkbuf[pl.ds(gg * 16, 16)]
                    offv = pkv >> 19
                    for j in range(16):
                        gp = basepos + gg * 16 + j
                        ok = jnp.logical_and(gp >= s_s, gp < e_s)

                        @pl.when(ok)
                        def _upd(offv=offv, j=j, gg=gg):
                            arow = accum.at[offv[j]]
                            mrow = mbuf.at[gg * 16 + j]
                            for cc in range(HID // 16):
                                sl = pl.ds(cc * 16, 16)
                                arow[sl] = jnp.maximum(arow[sl], mrow[sl])
                return c2

            lax.fori_loop(0, npc, piece, 0)
            return c

        lax.fori_loop(0, NWORK, per_w, 0)
        pltpu.sync_copy(accum, out_hbm.at[pl.ds(o * NPW, NPW)])

    return k(M, routed, ends)


def _prep_body(x_ref, wp_ref, bp_ref, wad_ref, was_ref, ad_ref, as_ref):
    h = jnp.maximum(jnp.dot(x_ref[...], wp_ref[...],
                            preferred_element_type=jnp.float32) + bp_ref[...], 0.0)
    ad_ref[...] = jnp.dot(h, wad_ref[...], preferred_element_type=jnp.float32)
    as_ref[...] = jnp.dot(h, was_ref[...], preferred_element_type=jnp.float32)


def _prep(x, Wp, bp, Wad, Was):
    grid = (NPAD // NBLK,)
    return pl.pallas_call(
        _prep_body,
        grid=grid,
        in_specs=[
            pl.BlockSpec((NBLK, 128), lambda i: (i, 0)),
            pl.BlockSpec((128, HID), lambda i: (0, 0)),
            pl.BlockSpec((1, HID), lambda i: (0, 0)),
            pl.BlockSpec((HID, HID), lambda i: (0, 0)),
            pl.BlockSpec((HID, HID), lambda i: (0, 0)),
        ],
        out_specs=[
            pl.BlockSpec((NBLK, HID), lambda i: (i, 0)),
            pl.BlockSpec((NBLK, HID), lambda i: (i, 0)),
        ],
        out_shape=[
            jax.ShapeDtypeStruct((NPAD, HID), jnp.float32),
            jax.ShapeDtypeStruct((NPAD, HID), jnp.float32),
        ],
    )(x, Wp, bp, Wad, Was)


def _edge_mlp_body(g_ref, e_ref, wae_ref, ba_ref, wb_ref, bb_ref, o_ref):
    pre = g_ref[...] + jnp.dot(e_ref[...], wae_ref[...],
                               preferred_element_type=jnp.float32) + ba_ref[...]
    o_ref[...] = jnp.dot(jnp.maximum(pre, 0.0), wb_ref[...],
                         preferred_element_type=jnp.float32) + bb_ref[...]


def _edge_mlp(G, eattr, Wae, ba, Wb, bb):
    ne = G.shape[0]
    grid = (ne // EBLK,)
    return pl.pallas_call(
        _edge_mlp_body,
        grid=grid,
        in_specs=[
            pl.BlockSpec((EBLK, HID), lambda i: (i, 0)),
            pl.BlockSpec((EBLK, EDIM), lambda i: (i, 0)),
            pl.BlockSpec((EDIM, HID), lambda i: (0, 0)),
            pl.BlockSpec((1, HID), lambda i: (0, 0)),
            pl.BlockSpec((HID, HID), lambda i: (0, 0)),
            pl.BlockSpec((1, HID), lambda i: (0, 0)),
        ],
        out_specs=pl.BlockSpec((EBLK, HID), lambda i: (i, 0)),
        out_shape=jax.ShapeDtypeStruct((ne, HID), jnp.float32),
    )(G, eattr, Wae, ba, Wb, bb)


def _bn_prep_body(agg_ref, g_ref, be_ref, wad_ref, was_ref, ad_ref, as_ref):
    a = agg_ref[...]
    a = jnp.where(jnp.isfinite(a), a, 0.0)
    row = jax.lax.broadcasted_iota(jnp.int32, (NPAD, 1), 0)
    am = jnp.where(row < N, a, 0.0)
    mu = jnp.sum(am, axis=0, keepdims=True) / N
    var = jnp.sum(am * am, axis=0, keepdims=True) / N - mu * mu
    h = jnp.maximum(g_ref[...] * (a - mu) * jax.lax.rsqrt(var + EPS) + be_ref[...], 0.0)
    ad_ref[...] = jnp.dot(h, wad_ref[...], preferred_element_type=jnp.float32)
    as_ref[...] = jnp.dot(h, was_ref[...], preferred_element_type=jnp.float32)


def _bn_prep(agg, g, be, Wad, Was):
    return pl.pallas_call(
        _bn_prep_body,
        out_shape=[
            jax.ShapeDtypeStruct((NPAD, HID), jnp.float32),
            jax.ShapeDtypeStruct((NPAD, HID), jnp.float32),
        ],
    )(agg, g, be, Wad, Was)


def _final_body(agg_ref, g_ref, be_ref, wm1_ref, bm1_ref, wm2_ref, bm2_ref, o_ref):
    a = agg_ref[...]
    a = jnp.where(jnp.isfinite(a), a, 0.0)
    row = jax.lax.broadcasted_iota(jnp.int32, (NPAD, 1), 0)
    am = jnp.where(row < N, a, 0.0)
    mu = jnp.sum(am, axis=0, keepdims=True) / N
    var = jnp.sum(am * am, axis=0, keepdims=True) / N - mu * mu
    h = jnp.maximum(g_ref[...] * (a - mu) * jax.lax.rsqrt(var + EPS) + be_ref[...], 0.0)
    t = jnp.maximum(jnp.dot(h, wm1_ref[...], preferred_element_type=jnp.float32) + bm1_ref[...], 0.0)
    o_ref[...] = jnp.dot(t, wm2_ref[...], preferred_element_type=jnp.float32) + bm2_ref[...]


def _final(agg, g, be, Wm1, bm1, Wm2, bm2):
    return pl.pallas_call(
        _final_body,
        out_shape=jax.ShapeDtypeStruct((NPAD, HID), jnp.float32),
    )(agg, g, be, Wm1, bm1, Wm2, bm2)


def kernel(x, edge_index, edge_attr, Wp, bp, W0a, b0a, W0b, b0b, g0, be0,
           W1a, b1a, W1b, b1b, g1, be1, Wm1, bm1, Wm2, bm2):
    src = edge_index[0]
    dst = edge_index[1]

    xpad = jnp.pad(x, ((0, NPAD - N), (0, 0)))
    bp2 = bp.reshape(1, HID)

    dst_pad = jnp.pad(dst, (0, EPAD - E), constant_values=NPAD - 1)
    src_pad = jnp.pad(src, (0, EPAD - E), constant_values=NPAD - 1)
    dst2d = dst_pad.reshape(EPAD // CH, CH)
    src2d = src_pad.reshape(EPAD // CH, CH)
    eattr_pad = jnp.pad(edge_attr, ((0, EPAD - E), (0, 0)))

    W0ad, W0as, W0ae = W0a[:HID], W0a[HID:2 * HID], W0a[2 * HID:]
    W1ad, W1as, W1ae = W1a[:HID], W1a[HID:2 * HID], W1a[2 * HID:]

    routed, ends = _sc_route(dst2d)

    Ad0, As0 = _prep(xpad, Wp, bp2, W0ad, W0as)
    G0 = _sc_gather_add(Ad0, As0, dst2d, src2d)
    M0 = _edge_mlp(G0, eattr_pad, W0ae, b0a.reshape(1, HID), W0b, b0b.reshape(1, HID))
    agg0 = _sc_segmax(M0, routed, ends)

    Ad1, As1 = _bn_prep(agg0, g0.reshape(1, HID), be0.reshape(1, HID), W1ad, W1as)
    G1 = _sc_gather_add(Ad1, As1, dst2d, src2d)
    M1 = _edge_mlp(G1, eattr_pad, W1ae, b1a.reshape(1, HID), W1b, b1b.reshape(1, HID))
    agg1 = _sc_segmax(M1, routed, ends)

    out = _final(agg1, g1.reshape(1, HID), be1.reshape(1, HID),
                 Wm1, bm1.reshape(1, HID), Wm2, bm2.reshape(1, HID))
    return out[:N]


# speedup vs baseline: 1.0809x; 1.0809x over previous
import functools

import jax
import jax.numpy as jnp
from jax import lax
from jax.experimental import pallas as pl
from jax.experimental.pallas import tpu as pltpu
from jax.experimental.pallas import tpu_sc as plsc

N = 10000
NPAD = 10240
E = 320000
HID = 64
EDIM = 16
EPS = 1e-5

NBLK = 128
EBLK = 512

NWORK = 32
CH = 128
CPW = 80
EPAD = NWORK * CPW * CH


def _sc_gather_add(Ad, As, dst2d, src2d):
    mesh = plsc.VectorSubcoreMesh(core_axis_name="c", subcore_axis_name="s")

    @functools.partial(
        pl.kernel,
        out_type=jax.ShapeDtypeStruct((EPAD, HID), jnp.float32),
        mesh=mesh,
        scratch_types=[
            pltpu.VMEM((CPW, CH), jnp.int32),
            pltpu.VMEM((CPW, CH), jnp.int32),
            pltpu.VMEM((2, CH, HID), jnp.float32),
            pltpu.VMEM((2, CH, HID), jnp.float32),
            pltpu.SemaphoreType.DMA((2,)),
            pltpu.SemaphoreType.DMA((2,)),
        ],
        compiler_params=pltpu.CompilerParams(use_tc_tiling_on_sc=False),
    )
    def k(ad_hbm, as_hbm, d_hbm, s_hbm, out_hbm, didx, sidx, bufa, bufb,
          sema, semb):
        wid = lax.axis_index("s") * 2 + lax.axis_index("c")
        row0 = wid * CPW
        pltpu.sync_copy(d_hbm.at[pl.ds(row0, CPW)], didx)
        pltpu.sync_copy(s_hbm.at[pl.ds(row0, CPW)], sidx)

        def issue(j, slot):
            ca = pltpu.async_copy(ad_hbm.at[didx.at[j]], bufa.at[slot],
                                  sema.at[slot])
            cb = pltpu.async_copy(as_hbm.at[sidx.at[j]], bufb.at[slot],
                                  semb.at[slot])
            return ca, cb

        def drain(j, slot):
            pltpu.make_async_copy(ad_hbm.at[didx.at[j]], bufa.at[slot],
                                  sema.at[slot]).wait()
            pltpu.make_async_copy(as_hbm.at[sidx.at[j]], bufb.at[slot],
                                  semb.at[slot]).wait()

        def process(j, slot):
            ba = bufa.at[slot]
            bb = bufb.at[slot]

            def add_body(i, c2):
                ra = ba.at[i]
                rb = bb.at[i]
                for c in range(HID // 16):
                    sl = pl.ds(c * 16, 16)
                    ra[sl] = ra[sl] + rb[sl]
                return c2

            lax.fori_loop(0, CH, add_body, 0, unroll=4)
            pltpu.sync_copy(ba, out_hbm.at[pl.ds((row0 + j) * CH, CH)])

        issue(0, 0)
        issue(1, 1)

        def chunk_body(j2, carry):
            j = j2 * 2
            drain(j, 0)
            process(j, 0)

            @pl.when(j2 + 1 < CPW // 2)
            def _():
                issue(j + 2, 0)

            drain(j + 1, 1)
            process(j + 1, 1)

            @pl.when(j2 + 1 < CPW // 2)
            def _():
                issue(j + 3, 1)

            return carry

        lax.fori_loop(0, CPW // 2, chunk_body, 0)

    return k(Ad, As, dst2d, src2d)


NPW = 320
EPW = EPAD // NWORK
PIECE = 512
RROW = EPW + PIECE
_MAGIC = 209716


def _sc_route(dst2d):
    mesh = plsc.VectorSubcoreMesh(core_axis_name="c", subcore_axis_name="s")

    @functools.partial(
        pl.kernel,
        out_type=[
            jax.ShapeDtypeStruct((NWORK * RROW,), jnp.int32),
            jax.ShapeDtypeStruct((NWORK * NWORK,), jnp.int32),
        ],
        mesh=mesh,
        scratch_types=[
            pltpu.VMEM((CPW, CH), jnp.int32),
            pltpu.VMEM((EPW,), jnp.int32),
            pltpu.VMEM((NWORK,), jnp.int32),
            pltpu.VMEM((NWORK,), jnp.int32),
            pltpu.VMEM((NWORK,), jnp.int32),
        ],
        compiler_params=pltpu.CompilerParams(needs_layout_passes=False),
    )
    def k(d_hbm, routed_hbm, ends_hbm, didx, pend, cnt, cur, endsv):
        wid = lax.axis_index("s") * 2 + lax.axis_index("c")
        pltpu.sync_copy(d_hbm.at[pl.ds(wid * CPW, CPW)], didx)
        beta, _ = plsc.scan_count(lax.iota(jnp.int32, 16))
        lanes = lax.iota(jnp.int32, 16)
        lane0m = lanes == 0
        zeros = jnp.zeros((16,), jnp.int32)
        cnt[pl.ds(0, 16)] = zeros
        cnt[pl.ds(16, 16)] = zeros

        def count_row(r, c):
            drow = didx.at[r]
            for gg in range(CH // 16):
                v = drow[pl.ds(gg * 16, 16)]
                b = (v * _MAGIC) >> 26
                rank, last = plsc.scan_count(b)
                base = plsc.load_gather(cnt, [b])
                plsc.store_scatter(cnt, [b], base + (rank - beta) + 1,
                                   mask=last)
            return c

        lax.fori_loop(0, CPW, count_row, 0)

        c0 = cnt[pl.ds(0, 16)]
        c1 = cnt[pl.ds(16, 16)]
        run = jnp.int32(0)
        for kk in range(NWORK):
            ck = c0[kk] if kk < 16 else c1[kk - 16]
            kv = jnp.full((16,), kk, jnp.int32)
            plsc.store_scatter(cur, [kv], jnp.full((16,), run, jnp.int32),
                               mask=lane0m)
            run = run + ck
            plsc.store_scatter(endsv, [kv], jnp.full((16,), run, jnp.int32),
                               mask=lane0m)
        pltpu.sync_copy(
            endsv, ends_hbm.at[pl.ds(pl.multiple_of(wid * NWORK, 8), NWORK)])

        def place_row(r, c):
            drow = didx.at[r]
            for gg in range(CH // 16):
                v = drow[pl.ds(gg * 16, 16)]
                b = (v * _MAGIC) >> 26
                off = v - b * NPW
                eid = (wid * CPW + r) * CH + gg * 16 + lanes
                pk = (off << 19) | eid
                rank, last = plsc.scan_count(b)
                base = plsc.load_gather(cur, [b])
                pos = base + (rank - beta)
                plsc.store_scatter(pend, [pos], pk)
                plsc.store_scatter(cur, [b], pos + 1, mask=last)
            return c

        lax.fori_loop(0, CPW, place_row, 0)
        pltpu.sync_copy(
            pend, routed_hbm.at[pl.ds(pl.multiple_of(wid * RROW, 8), EPW)])

    return k(dst2d)


def _sc_segmax(M, routed, ends):
    mesh = plsc.VectorSubcoreMesh(core_axis_name="c", subcore_axis_name="s")

    @functools.partial(
        pl.kernel,
        out_type=jax.ShapeDtypeStruct((NPAD, HID), jnp.float32),
        mesh=mesh,
        scratch_types=[
            pltpu.VMEM((NPW + 8, HID), jnp.float32),
            pltpu.VMEM((PIECE, HID), jnp.float32),
            pltpu.VMEM((PIECE,), jnp.int32),
            pltpu.VMEM((PIECE,), jnp.int32),
            pltpu.VMEM((NWORK * NWORK,), jnp.int32),
            pltpu.SemaphoreType.DMA,
        ],
        compiler_params=pltpu.CompilerParams(
            needs_layout_passes=False, use_tc_tiling_on_sc=False),
    )
    def k(m_hbm, routed_hbm, ends_hbm, out_hbm, accum, mbuf, pkbuf, eidbuf,
          endsv, sem):
        o = lax.axis_index("s") * 2 + lax.axis_index("c")
        pltpu.sync_copy(ends_hbm, endsv)
        ninf = jnp.full((16,), -jnp.inf, jnp.float32)
        lanes = lax.iota(jnp.int32, 16)

        def init(r, c):
            arow = accum.at[r]
            for cc in range(HID // 16):
                arow[pl.ds(cc * 16, 16)] = ninf
            return c

        lax.fori_loop(0, NPW + 8, init, 0)
        ov = jnp.full((16,), o, jnp.int32)
        ovm1 = jnp.full((16,), jnp.maximum(o - 1, 0), jnp.int32)

        def per_w(w, c):
            wv = jnp.full((16,), w * NWORK, jnp.int32)
            e_s = plsc.load_gather(endsv, [wv + ov])[0]
            s_raw = plsc.load_gather(endsv, [wv + ovm1])[0]
            s_s = jnp.where(o > 0, s_raw, 0)
            c_s = e_s - s_s
            s8 = s_s & (-8)
            npc = (s_s - s8 + c_s + PIECE - 1) // PIECE

            def piece(p, c2):
                basepos = s8 + p * PIECE
                pltpu.sync_copy(
                    routed_hbm.at[
                        pl.ds(pl.multiple_of(w * RROW + basepos, 8), PIECE)],
                    pkbuf)

                def build(gg, c3):
                    sl = pl.ds(pl.multiple_of(gg * 16, 8), 16)
                    pkv = pkbuf[sl]
                    eidbuf[sl] = jnp.minimum(pkv & 0x7FFFF, EPAD - 1)
                    return c3

                lax.fori_loop(0, PIECE // 16, build, 0, unroll=4)
                nq = jnp.clip((e_s - basepos + 127) // 128, 0, PIECE // 128)

                def issue(q, c3):
                    pltpu.async_copy(
                        m_hbm.at[eidbuf.at[pl.ds(pl.multiple_of(q * 128, 8),
                                                 128)]],
                        mbuf.at[pl.ds(pl.multiple_of(q * 128, 8), 128)], sem)
                    return c3

                lax.fori_loop(0, nq, issue, 0)

                def drain(q, c3):
                    pltpu.make_async_copy(
                        m_hbm.at[eidbuf.at[pl.ds(pl.multiple_of(q * 128, 8),
                                                 128)]],
                        mbuf.at[pl.ds(pl.multiple_of(q * 128, 8), 128)],
                        sem).wait()
                    return c3

                lax.fori_loop(0, nq, drain, 0)

                def upd_grp(gg, c3):
                    base16 = pl.multiple_of(gg * 16, 8)
                    gidx = basepos + gg * 16 + lanes
                    pkv = pkbuf[pl.ds(base16, 16)]
                    validv = jnp.logical_and(gidx >= s_s, gidx < e_s)
                    offv = jnp.where(validv, pkv >> 19, NPW)
                    for j in range(16):
                        arow = accum.at[offv[j]]
                        mrow = mbuf.at[gg * 16 + j]
                        for cc in range(HID // 16):
                            sl = pl.ds(cc * 16, 16)
                            arow[sl] = jnp.maximum(arow[sl], mrow[sl])
                    return c3

                lax.fori_loop(0, PIECE // 16, upd_grp, 0)
                return c2

            lax.fori_loop(0, npc, piece, 0)
            return c

        lax.fori_loop(0, NWORK, per_w, 0)
        pltpu.sync_copy(accum.at[pl.ds(0, NPW)], out_hbm.at[pl.ds(o * NPW, NPW)])

    return k(M, routed, ends)


def _prep_body(x_ref, wp_ref, bp_ref, wad_ref, was_ref, ad_ref, as_ref):
    h = jnp.maximum(jnp.dot(x_ref[...], wp_ref[...],
                            preferred_element_type=jnp.float32) + bp_ref[...], 0.0)
    ad_ref[...] = jnp.dot(h, wad_ref[...], preferred_element_type=jnp.float32)
    as_ref[...] = jnp.dot(h, was_ref[...], preferred_element_type=jnp.float32)


def _prep(x, Wp, bp, Wad, Was):
    grid = (NPAD // NBLK,)
    return pl.pallas_call(
        _prep_body,
        grid=grid,
        in_specs=[
            pl.BlockSpec((NBLK, 128), lambda i: (i, 0)),
            pl.BlockSpec((128, HID), lambda i: (0, 0)),
            pl.BlockSpec((1, HID), lambda i: (0, 0)),
            pl.BlockSpec((HID, HID), lambda i: (0, 0)),
            pl.BlockSpec((HID, HID), lambda i: (0, 0)),
        ],
        out_specs=[
            pl.BlockSpec((NBLK, HID), lambda i: (i, 0)),
            pl.BlockSpec((NBLK, HID), lambda i: (i, 0)),
        ],
        out_shape=[
            jax.ShapeDtypeStruct((NPAD, HID), jnp.float32),
            jax.ShapeDtypeStruct((NPAD, HID), jnp.float32),
        ],
    )(x, Wp, bp, Wad, Was)


def _edge_mlp_body(g_ref, e_ref, wae_ref, ba_ref, wb_ref, bb_ref, o_ref):
    pre = g_ref[...] + jnp.dot(e_ref[...], wae_ref[...],
                               preferred_element_type=jnp.float32) + ba_ref[...]
    o_ref[...] = jnp.dot(jnp.maximum(pre, 0.0), wb_ref[...],
                         preferred_element_type=jnp.float32) + bb_ref[...]


def _edge_mlp(G, eattr, Wae, ba, Wb, bb):
    ne = G.shape[0]
    grid = (ne // EBLK,)
    return pl.pallas_call(
        _edge_mlp_body,
        grid=grid,
        in_specs=[
            pl.BlockSpec((EBLK, HID), lambda i: (i, 0)),
            pl.BlockSpec((EBLK, EDIM), lambda i: (i, 0)),
            pl.BlockSpec((EDIM, HID), lambda i: (0, 0)),
            pl.BlockSpec((1, HID), lambda i: (0, 0)),
            pl.BlockSpec((HID, HID), lambda i: (0, 0)),
            pl.BlockSpec((1, HID), lambda i: (0, 0)),
        ],
        out_specs=pl.BlockSpec((EBLK, HID), lambda i: (i, 0)),
        out_shape=jax.ShapeDtypeStruct((ne, HID), jnp.float32),
    )(G, eattr, Wae, ba, Wb, bb)


def _bn_prep_body(agg_ref, g_ref, be_ref, wad_ref, was_ref, ad_ref, as_ref):
    a = agg_ref[...]
    a = jnp.where(jnp.isfinite(a), a, 0.0)
    row = jax.lax.broadcasted_iota(jnp.int32, (NPAD, 1), 0)
    am = jnp.where(row < N, a, 0.0)
    mu = jnp.sum(am, axis=0, keepdims=True) / N
    var = jnp.sum(am * am, axis=0, keepdims=True) / N - mu * mu
    h = jnp.maximum(g_ref[...] * (a - mu) * jax.lax.rsqrt(var + EPS) + be_ref[...], 0.0)
    ad_ref[...] = jnp.dot(h, wad_ref[...], preferred_element_type=jnp.float32)
    as_ref[...] = jnp.dot(h, was_ref[...], preferred_element_type=jnp.float32)


def _bn_prep(agg, g, be, Wad, Was):
    return pl.pallas_call(
        _bn_prep_body,
        out_shape=[
            jax.ShapeDtypeStruct((NPAD, HID), jnp.float32),
            jax.ShapeDtypeStruct((NPAD, HID), jnp.float32),
        ],
    )(agg, g, be, Wad, Was)


def _final_body(agg_ref, g_ref, be_ref, wm1_ref, bm1_ref, wm2_ref, bm2_ref, o_ref):
    a = agg_ref[...]
    a = jnp.where(jnp.isfinite(a), a, 0.0)
    row = jax.lax.broadcasted_iota(jnp.int32, (NPAD, 1), 0)
    am = jnp.where(row < N, a, 0.0)
    mu = jnp.sum(am, axis=0, keepdims=True) / N
    var = jnp.sum(am * am, axis=0, keepdims=True) / N - mu * mu
    h = jnp.maximum(g_ref[...] * (a - mu) * jax.lax.rsqrt(var + EPS) + be_ref[...], 0.0)
    t = jnp.maximum(jnp.dot(h, wm1_ref[...], preferred_element_type=jnp.float32) + bm1_ref[...], 0.0)
    o_ref[...] = jnp.dot(t, wm2_ref[...], preferred_element_type=jnp.float32) + bm2_ref[...]


def _final(agg, g, be, Wm1, bm1, Wm2, bm2):
    return pl.pallas_call(
        _final_body,
        out_shape=jax.ShapeDtypeStruct((NPAD, HID), jnp.float32),
    )(agg, g, be, Wm1, bm1, Wm2, bm2)


def kernel(x, edge_index, edge_attr, Wp, bp, W0a, b0a, W0b, b0b, g0, be0,
           W1a, b1a, W1b, b1b, g1, be1, Wm1, bm1, Wm2, bm2):
    src = edge_index[0]
    dst = edge_index[1]

    xpad = jnp.pad(x, ((0, NPAD - N), (0, 0)))
    bp2 = bp.reshape(1, HID)

    dst_pad = jnp.pad(dst, (0, EPAD - E), constant_values=NPAD - 1)
    src_pad = jnp.pad(src, (0, EPAD - E), constant_values=NPAD - 1)
    dst2d = dst_pad.reshape(EPAD // CH, CH)
    src2d = src_pad.reshape(EPAD // CH, CH)
    eattr_pad = jnp.pad(edge_attr, ((0, EPAD - E), (0, 0)))

    W0ad, W0as, W0ae = W0a[:HID], W0a[HID:2 * HID], W0a[2 * HID:]
    W1ad, W1as, W1ae = W1a[:HID], W1a[HID:2 * HID], W1a[2 * HID:]

    routed, ends = _sc_route(dst2d)

    Ad0, As0 = _prep(xpad, Wp, bp2, W0ad, W0as)
    G0 = _sc_gather_add(Ad0, As0, dst2d, src2d)
    M0 = _edge_mlp(G0, eattr_pad, W0ae, b0a.reshape(1, HID), W0b, b0b.reshape(1, HID))
    agg0 = _sc_segmax(M0, routed, ends)

    Ad1, As1 = _bn_prep(agg0, g0.reshape(1, HID), be0.reshape(1, HID), W1ad, W1as)
    G1 = _sc_gather_add(Ad1, As1, dst2d, src2d)
    M1 = _edge_mlp(G1, eattr_pad, W1ae, b1a.reshape(1, HID), W1b, b1b.reshape(1, HID))
    agg1 = _sc_segmax(M1, routed, ends)

    out = _final(agg1, g1.reshape(1, HID), be1.reshape(1, HID),
                 Wm1, bm1.reshape(1, HID), Wm2, bm2.reshape(1, HID))
    return out[:N]
